# Initial kernel scaffold; baseline (speedup 1.0000x reference)
import jax
import jax.numpy as jnp
from jax.experimental import pallas as pl


def kernel(indices, x_embed):
    raise NotImplementedError("write your pallas kernel here")


# speedup vs baseline: 3.0193x; 3.0193x over previous
import functools

import jax
import jax.numpy as jnp
from jax import lax
from jax.experimental import pallas as pl
from jax.experimental.pallas import tpu as pltpu
from jax.experimental.pallas import tpu_sc as plsc

NC = 2
NS = 16
NW = NC * NS
CH = 128


def kernel(indices, x_embed):
    out_shape = indices.shape + x_embed.shape[-1:]
    D = x_embed.shape[-1]
    B = indices.size
    b_per_w = B // NW
    n_chunks = b_per_w // CH

    idx = indices.reshape(NW, n_chunks, CH).astype(jnp.int32)

    mesh = plsc.VectorSubcoreMesh(
        core_axis_name="c", subcore_axis_name="s",
        num_cores=NC, num_subcores=NS)

    @functools.partial(
        pl.kernel,
        out_type=jax.ShapeDtypeStruct((B, D), jnp.float32),
        mesh=mesh,
        compiler_params=pltpu.CompilerParams(use_tc_tiling_on_sc=False),
        scratch_types=[
            pltpu.VMEM((n_chunks, CH), jnp.int32),
            pltpu.VMEM((2, CH, D), jnp.float32),
            pltpu.SemaphoreType.DMA,
            pltpu.SemaphoreType.DMA,
        ],
    )
    def gather_kernel(idx_hbm, table_hbm, out_hbm, idx_v, rows_v, sem0, sem1):
        wid = lax.axis_index("s") * NC + lax.axis_index("c")
        base = wid * b_per_w
        sems = (sem0, sem1)
        pltpu.sync_copy(idx_hbm.at[wid], idx_v)
        for j in range(min(2, n_chunks)):
            pltpu.async_copy(table_hbm.at[idx_v.at[j]], rows_v.at[j % 2],
                             sems[j % 2])
        for j in range(n_chunks):
            buf = j % 2
            pltpu.make_async_copy(table_hbm.at[idx_v.at[j]], rows_v.at[buf],
                                  sems[buf]).wait()
            pltpu.sync_copy(rows_v.at[buf],
                            out_hbm.at[pl.ds(base + j * CH, CH)])
            if j + 2 < n_chunks:
                pltpu.async_copy(table_hbm.at[idx_v.at[j + 2]],
                                 rows_v.at[buf], sems[buf])

    out = gather_kernel(idx, x_embed)
    return out.reshape(out_shape)


# speedup vs baseline: 4.2181x; 1.3971x over previous
import functools

import jax
import jax.numpy as jnp
from jax import lax
from jax.experimental import pallas as pl
from jax.experimental.pallas import tpu as pltpu
from jax.experimental.pallas import tpu_sc as plsc

NC, NS = 2, 16
NW = NC * NS
NIMG, NTOK, D = 64, 1024, 64
IG = 8
DG = 4
IPW = NIMG // IG
DPW = D // DG


def kernel(indices, x_embed):
    idx_flat = indices.reshape(-1).astype(jnp.int32)
    tt_flat = x_embed.T.reshape(-1)

    mesh = plsc.VectorSubcoreMesh(
        core_axis_name="c", subcore_axis_name="s",
        num_cores=NC, num_subcores=NS)

    @functools.partial(
        pl.kernel,
        out_type=jax.ShapeDtypeStruct((NIMG, D, NTOK), jnp.float32),
        mesh=mesh,
        compiler_params=pltpu.CompilerParams(
            use_tc_tiling_on_sc=True, needs_layout_passes=False),
        scratch_types=[
            pltpu.VMEM((IPW * NTOK,), jnp.int32),
            pltpu.VMEM((16384,), jnp.float32),
            pltpu.VMEM((2, DPW, NTOK), jnp.float32),
            pltpu.SemaphoreType.DMA,
        ],
    )
    def tgather(idx_hbm, tt_hbm, out_hbm, idx_v, tt_v, ob, sem):
        wid = lax.axis_index("s") * NC + lax.axis_index("c")
        ig = wid % IG
        dg = wid // IG
        pltpu.sync_copy(idx_hbm.at[pl.ds(ig * IPW * NTOK, IPW * NTOK)], idx_v)
        pltpu.sync_copy(tt_hbm.at[pl.ds(dg * DPW * NTOK, DPW * NTOK)], tt_v)

        def do_image(im, buf):
            def body(g, _):
                iv = idx_v[pl.ds(im * NTOK + g * 16, 16)]
                for dd in range(DPW):
                    val = plsc.load_gather(tt_v, [iv + dd * NTOK])
                    ob[buf, dd, pl.ds(g * 16, 16)] = val
                return 0
            lax.fori_loop(0, NTOK // 16, body, 0)

        for im in range(IPW):
            buf = im % 2
            if im >= 2:
                pltpu.make_async_copy(
                    ob.at[buf],
                    out_hbm.at[ig * IPW + im - 2,
                               pl.ds(dg * DPW, DPW), :], sem).wait()
            do_image(im, buf)
            pltpu.async_copy(
                ob.at[buf],
                out_hbm.at[ig * IPW + im, pl.ds(dg * DPW, DPW), :], sem)
        for im in range(IPW - 2, IPW):
            buf = im % 2
            pltpu.make_async_copy(
                ob.at[buf],
                out_hbm.at[ig * IPW + im, pl.ds(dg * DPW, DPW), :], sem).wait()

    out = tgather(idx_flat, tt_flat)
    return jnp.transpose(out, (0, 2, 1))


# speedup vs baseline: 7.1000x; 1.6832x over previous
import functools

import jax
import jax.numpy as jnp
from jax import lax
from jax.experimental import pallas as pl
from jax.experimental.pallas import tpu as pltpu
from jax.experimental.pallas import tpu_sc as plsc

NC, NS = 2, 16
NW = NC * NS
NIMG, NTOK, D = 64, 1024, 64
IG = 8
DG = 4
IPW = NIMG // IG
DPW = D // DG


def kernel(indices, x_embed):
    idx_flat = indices.reshape(-1).astype(jnp.int32)
    tt_flat = x_embed.T.reshape(-1)

    mesh = plsc.VectorSubcoreMesh(
        core_axis_name="c", subcore_axis_name="s",
        num_cores=NC, num_subcores=NS)

    @functools.partial(
        pl.kernel,
        out_type=jax.ShapeDtypeStruct((NIMG, D, NTOK), jnp.float32),
        mesh=mesh,
        compiler_params=pltpu.CompilerParams(
            use_tc_tiling_on_sc=True, needs_layout_passes=False),
        scratch_types=[
            pltpu.VMEM((IPW * NTOK,), jnp.int32),
            pltpu.VMEM((16384,), jnp.float32),
            pltpu.VMEM((2, DPW, NTOK), jnp.float32),
            pltpu.SemaphoreType.DMA,
        ],
    )
    def tgather(idx_hbm, tt_hbm, out_hbm, idx_v, tt_v, ob, sem):
        wid = lax.axis_index("s") * NC + lax.axis_index("c")
        ig = wid % IG
        dg = wid // IG
        pltpu.sync_copy(idx_hbm.at[pl.ds(ig * IPW * NTOK, IPW * NTOK)], idx_v)
        pltpu.sync_copy(tt_hbm.at[pl.ds(dg * DPW * NTOK, DPW * NTOK)], tt_v)

        def do_image(im, buf):
            @plsc.parallel_loop(0, NTOK // 16, unroll=2)
            def body(g):
                iv = idx_v[pl.ds(im * NTOK + g * 16, 16)]
                vals = [plsc.load_gather(tt_v, [iv + dd * NTOK])
                        for dd in range(DPW)]
                for dd in range(DPW):
                    ob[buf, dd, pl.ds(g * 16, 16)] = vals[dd]

        for im in range(IPW):
            buf = im % 2
            if im >= 2:
                pltpu.make_async_copy(
                    ob.at[buf],
                    out_hbm.at[ig * IPW + im - 2,
                               pl.ds(dg * DPW, DPW), :], sem).wait()
            do_image(im, buf)
            pltpu.async_copy(
                ob.at[buf],
                out_hbm.at[ig * IPW + im, pl.ds(dg * DPW, DPW), :], sem)
        for im in range(IPW - 2, IPW):
            buf = im % 2
            pltpu.make_async_copy(
                ob.at[buf],
                out_hbm.at[ig * IPW + im, pl.ds(dg * DPW, DPW), :], sem).wait()

    out = tgather(idx_flat, tt_flat)
    return jnp.transpose(out, (0, 2, 1))
